# baseline (speedup 1.0000x reference)
import functools

import jax
import jax.numpy as jnp
from jax import lax
from jax.experimental import pallas as pl
from jax.experimental.pallas import tpu as pltpu
from jax.experimental.pallas import tpu_sc as plsc

D_MODEL = 256
HEIGHT = 384
WIDTH = 384
NTOK = 16 * 2048
DM = D_MODEL // 2

NC, NS, LANES = 2, 16, 16
NW = NC * NS
TPW = NTOK // NW
CHUNK = 64
NCHUNK = TPW // CHUNK
SLOTS = 2 * CHUNK


def _sc_body(xf, cf, tab, out, coordv, idxv, xv, rowsv, sem):
    wid = lax.axis_index("s") * NC + lax.axis_index("c")
    tok0 = wid * TPW
    pltpu.sync_copy(cf.at[pl.ds(tok0 * 2, TPW * 2)], coordv)
    lane = lax.iota(jnp.int32, LANES)
    offs = (1 - (lane & 1)) * 384

    def chunk_body(c, carry):
        cbase = c * SLOTS
        for g in range(SLOTS // LANES):
            v = coordv[pl.ds(cbase + g * LANES, LANES)]
            idxv[pl.ds(g * LANES, LANES)] = (v / 100.0).astype(jnp.int32) + offs
        xbase = (tok0 + c * CHUNK) * D_MODEL
        pltpu.sync_copy(xf.at[pl.ds(xbase, CHUNK * D_MODEL)], xv)
        pltpu.async_copy(tab.at[idxv], rowsv, sem).wait()

        def add_body(r, acc):
            for k in range(DM // LANES):
                v = rowsv[r, pl.ds(k * LANES, LANES)]
                plsc.addupdate(xv.at[pl.ds(((r ^ 1) * (DM // LANES) + k) * LANES, LANES)], v)
            return acc

        lax.fori_loop(0, SLOTS, add_body, 0)
        pltpu.sync_copy(xv, out.at[pl.ds(xbase, CHUNK * D_MODEL)])
        return carry

    lax.fori_loop(0, NCHUNK, chunk_body, 0)


_sc_call = pl.kernel(
    _sc_body,
    out_type=jax.ShapeDtypeStruct((NTOK * D_MODEL,), jnp.float32),
    mesh=plsc.VectorSubcoreMesh(
        core_axis_name="c", subcore_axis_name="s",
        num_cores=NC, num_subcores=NS,
    ),
    scratch_types=[
        pltpu.VMEM((TPW * 2,), jnp.float32),
        pltpu.VMEM((SLOTS,), jnp.int32),
        pltpu.VMEM((CHUNK * D_MODEL,), jnp.float32),
        pltpu.VMEM((SLOTS, DM), jnp.float32),
        pltpu.SemaphoreType.DMA,
    ],
)


@jax.jit
def kernel(x, coord, pe):
    tw = pe[:DM, 0, :].T
    th = pe[DM:, :, 0].T
    tab = 0.1 * jnp.concatenate([tw, th], axis=0)
    outf = _sc_call(x.reshape(-1), coord.reshape(-1), tab)
    return outf.reshape(x.shape)


# speedup vs baseline: 1.6859x; 1.6859x over previous
import functools

import jax
import jax.numpy as jnp
from jax import lax
from jax.experimental import pallas as pl
from jax.experimental.pallas import tpu as pltpu
from jax.experimental.pallas import tpu_sc as plsc

D_MODEL = 256
HEIGHT = 384
WIDTH = 384
NTOK = 16 * 2048
DM = D_MODEL // 2

NC, NS, LANES = 2, 16, 16
NW = NC * NS
TPW = NTOK // NW
CHUNK = 64
NCHUNK = TPW // CHUNK
SLOTS = 2 * CHUNK
NBUF = 2


def _sc_body(x2, cf, tab, out, coordv, *bufs):
    sets = [bufs[i * 6:(i + 1) * 6] for i in range(NBUF)]
    wid = lax.axis_index("s") * NC + lax.axis_index("c")
    tok0 = wid * TPW
    pltpu.sync_copy(cf.at[pl.ds(tok0 * 2, TPW * 2)], coordv)
    lane = lax.iota(jnp.int32, LANES)
    offs = (1 - (lane & 1)) * 384

    def issue_in(c, S):
        idxv, xv, rowsv, sem_x, sem_g, _ = S
        cbase = c * SLOTS
        for g in range(SLOTS // LANES):
            v = coordv[pl.ds(cbase + g * LANES, LANES)]
            idxv[pl.ds(g * LANES, LANES)] = (v / 100.0).astype(jnp.int32) + offs
        hx = pltpu.async_copy(x2.at[pl.ds(tok0 + c * CHUNK, CHUNK), :], xv, sem_x)
        hg = pltpu.async_copy(tab.at[idxv], rowsv, sem_g)
        return hx, hg

    def accumulate(S):
        idxv, xv, rowsv, *_ = S

        def add_body(s, acc):
            cb = (1 - (s & 1)) * DM
            for k in range(DM // LANES):
                v = rowsv[s, pl.ds(k * LANES, LANES)]
                plsc.addupdate(xv.at[s >> 1, pl.ds(cb + k * LANES, LANES)], v)
            return acc

        lax.fori_loop(0, SLOTS, add_body, 0)

    inflight = {}
    pending_out = {}
    for c in range(NCHUNK):
        if c == 0:
            inflight[0] = issue_in(0, sets[0])
        if c + 1 < NCHUNK:
            if c >= 1:
                pending_out.pop(c - 1).wait()
            inflight[c + 1] = issue_in(c + 1, sets[(c + 1) % NBUF])
        hx, hg = inflight.pop(c)
        hx.wait()
        hg.wait()
        S = sets[c % NBUF]
        accumulate(S)
        pending_out[c] = pltpu.async_copy(
            S[1], out.at[pl.ds(tok0 + c * CHUNK, CHUNK), :], S[5])
    for c in sorted(pending_out):
        pending_out.pop(c).wait()


def _buf_set():
    return [
        pltpu.VMEM((SLOTS,), jnp.int32),
        pltpu.VMEM((CHUNK, D_MODEL), jnp.float32),
        pltpu.VMEM((SLOTS, DM), jnp.float32),
        pltpu.SemaphoreType.DMA,
        pltpu.SemaphoreType.DMA,
        pltpu.SemaphoreType.DMA,
    ]


_sc_call = pl.kernel(
    _sc_body,
    out_type=jax.ShapeDtypeStruct((NTOK, D_MODEL), jnp.float32),
    mesh=plsc.VectorSubcoreMesh(
        core_axis_name="c", subcore_axis_name="s",
        num_cores=NC, num_subcores=NS,
    ),
    scratch_types=[pltpu.VMEM((TPW * 2,), jnp.float32)]
    + _buf_set() + _buf_set(),
)


@jax.jit
def kernel(x, coord, pe):
    tw = pe[:DM, 0, :].T
    th = pe[DM:, :, 0].T
    tab = 0.1 * jnp.concatenate([tw, th], axis=0)
    out2 = _sc_call(x.reshape(NTOK, D_MODEL), coord.reshape(-1), tab)
    return out2.reshape(x.shape)
